# Initial kernel scaffold; baseline (speedup 1.0000x reference)
import jax
import jax.numpy as jnp
from jax.experimental import pallas as pl


def kernel(x, adj, W1, b1, W2, b2):
    raise NotImplementedError("write your pallas kernel here")


# speedup vs baseline: 6060.1584x; 6060.1584x over previous
import jax
import jax.numpy as jnp
from jax.experimental import pallas as pl


def _gcn_fused_kernel(x_ref, adj_ref, w1_ref, b1_ref, w2_ref, b2_ref, out_ref):
    adj = adj_ref[...]
    n = adj.shape[0]

    ones = jnp.ones((n, 1), jnp.float32)
    deg = jax.lax.dot_general(
        adj, ones, (((0,), (0,)), ((), ())),
        preferred_element_type=jnp.float32) + 1.0
    dinv = jnp.where(deg > 0, jax.lax.rsqrt(jnp.maximum(deg, 1e-12)), 0.0)
    d2 = dinv * dinv

    h = jnp.dot(x_ref[...], w1_ref[...], preferred_element_type=jnp.float32)
    y = jax.lax.dot_general(
        adj, dinv * h, (((0,), (0,)), ((), ())),
        preferred_element_type=jnp.float32)
    h1 = jnp.maximum(dinv * y + d2 * h + b1_ref[...], 0.0)

    h2 = jnp.dot(h1, w2_ref[...], preferred_element_type=jnp.float32)
    y2 = jax.lax.dot_general(
        adj, dinv * h2, (((0,), (0,)), ((), ())),
        preferred_element_type=jnp.float32)
    o = dinv * y2 + d2 * h2 + b2_ref[...]

    m = jnp.max(o, axis=1, keepdims=True)
    s = o - m
    lse = jnp.log(jnp.sum(jnp.exp(s), axis=1, keepdims=True))
    out_ref[...] = s - lse


def kernel(x, adj, W1, b1, W2, b2):
    n = x.shape[0]
    nclass = W2.shape[1]
    return pl.pallas_call(
        _gcn_fused_kernel,
        out_shape=jax.ShapeDtypeStruct((n, nclass), jnp.float32),
    )(x, adj, W1, b1.reshape(1, -1), W2, b2.reshape(1, -1))
